# baseline (speedup 1.0000x reference)
import functools

import jax
import jax.numpy as jnp
from jax import lax
from jax.experimental import pallas as pl
from jax.experimental.pallas import tpu as pltpu
from jax.experimental.pallas import tpu_sc as plsc

VOCAB = 1000000
EMB_DIM = 64
BATCH = 16384

_NC = 2
_NW = 32
_L = 16
_BPW = BATCH // _NW
_K = 16
_NT = VOCAB // 8


def _make_kernel():
    mesh = plsc.VectorSubcoreMesh(
        core_axis_name="c", subcore_axis_name="s", num_cores=_NC
    )

    @functools.partial(
        pl.kernel,
        mesh=mesh,
        out_type=jax.ShapeDtypeStruct((EMB_DIM, BATCH), jnp.float32),
        scratch_types=[
            pltpu.VMEM((_BPW + _L,), jnp.int32),
            pltpu.VMEM((EMB_DIM, _BPW), jnp.float32),
        ]
        + [pltpu.VMEM((1, 8, EMB_DIM), jnp.float32) for _ in range(_K)]
        + [pltpu.SemaphoreType.DMA for _ in range(_K)],
        compiler_params=pltpu.CompilerParams(
            use_tc_tiling_on_sc=True, needs_layout_passes=False
        ),
    )
    def tilegather(t3_hbm, idx_hbm, outT_hbm, idx_v, outT_v, *ring_and_sems):
        ring = ring_and_sems[:_K]
        sems = ring_and_sems[_K:]
        wid = lax.axis_index("s") * _NC + lax.axis_index("c")
        base = wid * _BPW
        iota = lax.iota(jnp.int32, _L)

        pltpu.sync_copy(idx_hbm.at[pl.ds(base, _BPW)], idx_v.at[pl.ds(0, _BPW)])

        def fire(w, k):
            pltpu.async_copy(t3_hbm.at[pl.ds(w >> 3, 1)], ring[k], sems[k])

        wv0 = idx_v[pl.ds(0, _L)]
        for k in range(_K):
            fire(wv0[k], k)

        def select(i_s, w, k):
            rv = jnp.full((_L,), 0, jnp.int32) + (w & 7)
            zv = jnp.zeros((_L,), jnp.int32)
            civ = jnp.full((_L,), 0, jnp.int32) + i_s
            for c in range(EMB_DIM // _L):
                ev = c * _L + iota
                vals = plsc.load_gather(ring[k], [zv, rv, ev])
                plsc.store_scatter(outT_v, [ev, civ], vals)

        def block(ib, wv_cur):
            wv_next = idx_v[pl.ds((ib + 1) * _L, _L)]
            for k in range(_K):
                pltpu.make_async_copy(
                    t3_hbm.at[pl.ds(0, 1)], ring[k], sems[k]
                ).wait()
                select(ib * _K + k, wv_cur[k], k)

                @pl.when(ib < _BPW // _K - 1)
                def _():
                    fire(wv_next[k], k)

            return wv_next

        lax.fori_loop(0, _BPW // _K, block, wv0)

        pltpu.sync_copy(outT_v, outT_hbm.at[:, pl.ds(base, _BPW)])

    return tilegather


_KERNEL = _make_kernel()


@jax.jit
def kernel(indices, table):
    t3 = table.reshape(_NT, 8, EMB_DIM)
    outT = _KERNEL(t3, indices.astype(jnp.int32))
    return outT.T


# speedup vs baseline: 1.0576x; 1.0576x over previous
import functools

import jax
import jax.numpy as jnp
from jax import lax
from jax.experimental import pallas as pl
from jax.experimental.pallas import tpu as pltpu
from jax.experimental.pallas import tpu_sc as plsc

VOCAB = 1000000
EMB_DIM = 64
BATCH = 16384

_NC = 2
_NW = 32
_L = 16
_BPW = BATCH // _NW
_R = 32
_NROUND = _BPW // _R
_NT = VOCAB // 8


def _make_kernel():
    mesh = plsc.VectorSubcoreMesh(
        core_axis_name="c", subcore_axis_name="s", num_cores=_NC
    )

    @functools.partial(
        pl.kernel,
        mesh=mesh,
        out_type=jax.ShapeDtypeStruct((EMB_DIM, BATCH), jnp.float32),
        scratch_types=[
            pltpu.VMEM((_BPW,), jnp.int32),
            pltpu.VMEM((EMB_DIM, _BPW), jnp.float32),
            pltpu.VMEM((_R, 8, EMB_DIM), jnp.float32),
            pltpu.VMEM((_R, 8, EMB_DIM), jnp.float32),
            pltpu.SemaphoreType.DMA,
            pltpu.SemaphoreType.DMA,
        ],
        compiler_params=pltpu.CompilerParams(
            use_tc_tiling_on_sc=True, needs_layout_passes=False
        ),
    )
    def tilegather(t3_hbm, idx_hbm, outT_hbm, idx_v, outT_v, tiles_a, tiles_b, sem_a, sem_b):
        wid = lax.axis_index("s") * _NC + lax.axis_index("c")
        base = wid * _BPW
        iota = lax.iota(jnp.int32, _L)

        pltpu.sync_copy(idx_hbm.at[pl.ds(base, _BPW)], idx_v)

        def fire_round(r, buf, sem):
            for half in range(_R // _L):
                wv = idx_v[pl.ds(r * _R + half * _L, _L)]
                for k in range(_L):
                    pltpu.async_copy(
                        t3_hbm.at[pl.ds(wv[k] >> 3, 1)],
                        buf.at[pl.ds(half * _L + k, 1)],
                        sem,
                    )

        def drain(buf, sem):
            pltpu.make_async_copy(t3_hbm.at[pl.ds(0, _R)], buf, sem).wait()

        def select_round(r, buf):
            for half in range(_R // _L):
                wv = idx_v[pl.ds(r * _R + half * _L, _L)]
                rv = wv & 7
                hv = half * _L + iota
                pos = r * _R + half * _L
                for e in range(EMB_DIM):
                    ev = jnp.full((_L,), 0, jnp.int32) + e
                    vals = plsc.load_gather(buf, [hv, rv, ev])
                    outT_v[e, pl.ds(pos, _L)] = vals

        fire_round(0, tiles_a, sem_a)
        fire_round(1, tiles_b, sem_b)

        def pair(rr, carry):
            r = rr * 2
            drain(tiles_a, sem_a)
            select_round(r, tiles_a)

            @pl.when(rr < _NROUND // 2 - 1)
            def _():
                fire_round(r + 2, tiles_a, sem_a)

            drain(tiles_b, sem_b)
            select_round(r + 1, tiles_b)

            @pl.when(rr < _NROUND // 2 - 1)
            def _():
                fire_round(r + 3, tiles_b, sem_b)

            return carry

        lax.fori_loop(0, _NROUND // 2, pair, 0)

        pltpu.sync_copy(outT_v, outT_hbm.at[:, pl.ds(base, _BPW)])

    return tilegather


_KERNEL = _make_kernel()


@jax.jit
def kernel(indices, table):
    t3 = table.reshape(_NT, 8, EMB_DIM)
    outT = _KERNEL(t3, indices.astype(jnp.int32))
    return outT.T
